# baseline (speedup 1.0000x reference)
import jax
import jax.numpy as jnp
from jax.experimental import pallas as pl
from jax.experimental.pallas import tpu as pltpu

_BM = 512
_BK = 2048


def _sage_block(adj_ref, x_ref, wrel_ref, wroot_ref, b_ref, out_ref,
                xbf_ref, agg_ref, deg_ref):
    i = pl.program_id(0)
    k = pl.program_id(1)
    nk = pl.num_programs(1)

    @pl.when(jnp.logical_and(i == 0, k == 0))
    def _cast_x():
        xbf_ref[...] = x_ref[...].astype(jnp.bfloat16)

    a = adj_ref[...]
    deg_p = jnp.sum(a, axis=1, keepdims=True)
    agg_p = jnp.dot(a.astype(jnp.bfloat16), xbf_ref[pl.ds(k * _BK, _BK), :],
                    preferred_element_type=jnp.float32)

    @pl.when(k == 0)
    def _init():
        agg_ref[...] = agg_p
        deg_ref[...] = deg_p

    @pl.when(k != 0)
    def _accum():
        agg_ref[...] += agg_p
        deg_ref[...] += deg_p

    @pl.when(k == nk - 1)
    def _finish():
        deg = jnp.clip(deg_ref[...], 1.0, None)
        agg = agg_ref[...] / deg
        x_blk = x_ref[pl.ds(i * _BM, _BM), :]
        out = (jnp.dot(agg, wrel_ref[...], preferred_element_type=jnp.float32)
               + jnp.dot(x_blk, wroot_ref[...],
                         preferred_element_type=jnp.float32)
               + b_ref[...])
        nrm = jnp.sqrt(jnp.sum(out * out, axis=1, keepdims=True))
        out = out / jnp.clip(nrm, 1e-12, None)
        out_ref[...] = jnp.where(out >= 0, out, 0.01 * out)


def kernel(x, adj, W_rel, W_root, b):
    B, N, C_in = x.shape
    C_out = W_rel.shape[1]
    x2 = x.reshape(N, C_in)
    adj2 = adj.reshape(N, N)
    b2 = b.reshape(1, C_out)
    out = pl.pallas_call(
        _sage_block,
        grid=(N // _BM, N // _BK),
        in_specs=[
            pl.BlockSpec((_BM, _BK), lambda i, k: (i, k)),
            pl.BlockSpec((N, C_in), lambda i, k: (0, 0)),
            pl.BlockSpec((C_in, C_out), lambda i, k: (0, 0)),
            pl.BlockSpec((C_in, C_out), lambda i, k: (0, 0)),
            pl.BlockSpec((1, C_out), lambda i, k: (0, 0)),
        ],
        out_specs=pl.BlockSpec((_BM, C_out), lambda i, k: (i, 0)),
        out_shape=jax.ShapeDtypeStruct((N, C_out), jnp.float32),
        scratch_shapes=[
            pltpu.VMEM((N, C_in), jnp.bfloat16),
            pltpu.VMEM((_BM, C_out), jnp.float32),
            pltpu.VMEM((_BM, 1), jnp.float32),
        ],
    )(adj2, x2, W_rel, W_root, b2)
    return out.reshape(B, N, C_out)


# speedup vs baseline: 1.2667x; 1.2667x over previous
import jax
import jax.numpy as jnp
from jax.experimental import pallas as pl
from jax.experimental.pallas import tpu as pltpu

_BM = 512


def _sage_block(adj_ref, x_ref, wrel_ref, wroot_ref, b_ref, out_ref, xbf_ref):
    i = pl.program_id(0)

    @pl.when(i == 0)
    def _cast_x():
        xbf_ref[...] = x_ref[...].astype(jnp.bfloat16)

    a = adj_ref[...]
    deg = jnp.clip(jnp.sum(a, axis=1, keepdims=True), 1.0, None)
    agg = jnp.dot(a.astype(jnp.bfloat16), xbf_ref[...],
                  preferred_element_type=jnp.float32)
    agg = agg / deg
    x_blk = x_ref[pl.ds(i * _BM, _BM), :]
    out = (jnp.dot(agg, wrel_ref[...], preferred_element_type=jnp.float32)
           + jnp.dot(x_blk, wroot_ref[...], preferred_element_type=jnp.float32)
           + b_ref[...])
    nrm = jnp.sqrt(jnp.sum(out * out, axis=1, keepdims=True))
    out = out / jnp.clip(nrm, 1e-12, None)
    out_ref[...] = jnp.where(out >= 0, out, 0.01 * out)


def kernel(x, adj, W_rel, W_root, b):
    B, N, C_in = x.shape
    C_out = W_rel.shape[1]
    x2 = x.reshape(N, C_in)
    adj2 = adj.reshape(N, N)
    b2 = b.reshape(1, C_out)
    out = pl.pallas_call(
        _sage_block,
        grid=(N // _BM,),
        in_specs=[
            pl.BlockSpec((_BM, N), lambda i: (i, 0)),
            pl.BlockSpec((N, C_in), lambda i: (0, 0)),
            pl.BlockSpec((C_in, C_out), lambda i: (0, 0)),
            pl.BlockSpec((C_in, C_out), lambda i: (0, 0)),
            pl.BlockSpec((1, C_out), lambda i: (0, 0)),
        ],
        out_specs=pl.BlockSpec((_BM, C_out), lambda i: (i, 0)),
        out_shape=jax.ShapeDtypeStruct((N, C_out), jnp.float32),
        scratch_shapes=[pltpu.VMEM((N, C_in), jnp.bfloat16)],
    )(adj2, x2, W_rel, W_root, b2)
    return out.reshape(B, N, C_out)


# speedup vs baseline: 1.2676x; 1.0007x over previous
import jax
import jax.numpy as jnp
from jax.experimental import pallas as pl
from jax.experimental.pallas import tpu as pltpu

_BM = 512


def _sage_block(adj_ref, x_ref, wrel_ref, wroot_ref, b_ref, out_ref):
    i = pl.program_id(0)
    a = adj_ref[...]
    deg = jnp.clip(jnp.sum(a, axis=1, keepdims=True), 1.0, None)
    agg = jnp.dot(a.astype(jnp.bfloat16), x_ref[...].astype(jnp.bfloat16),
                  preferred_element_type=jnp.float32)
    agg = agg / deg
    x_blk = x_ref[pl.ds(i * _BM, _BM), :]
    out = (jnp.dot(agg, wrel_ref[...], preferred_element_type=jnp.float32)
           + jnp.dot(x_blk, wroot_ref[...], preferred_element_type=jnp.float32)
           + b_ref[...])
    nrm = jnp.sqrt(jnp.sum(out * out, axis=1, keepdims=True))
    out = out / jnp.clip(nrm, 1e-12, None)
    out_ref[...] = jnp.where(out >= 0, out, 0.01 * out)


def kernel(x, adj, W_rel, W_root, b):
    B, N, C_in = x.shape
    C_out = W_rel.shape[1]
    x2 = x.reshape(N, C_in)
    adj2 = adj.reshape(N, N)
    b2 = b.reshape(1, C_out)
    out = pl.pallas_call(
        _sage_block,
        grid=(N // _BM,),
        in_specs=[
            pl.BlockSpec((_BM, N), lambda i: (i, 0)),
            pl.BlockSpec((N, C_in), lambda i: (0, 0)),
            pl.BlockSpec((C_in, C_out), lambda i: (0, 0)),
            pl.BlockSpec((C_in, C_out), lambda i: (0, 0)),
            pl.BlockSpec((1, C_out), lambda i: (0, 0)),
        ],
        out_specs=pl.BlockSpec((_BM, C_out), lambda i: (i, 0)),
        out_shape=jax.ShapeDtypeStruct((N, C_out), jnp.float32),
        compiler_params=pltpu.CompilerParams(
            dimension_semantics=("parallel",)),
    )(adj2, x2, W_rel, W_root, b2)
    return out.reshape(B, N, C_out)


# speedup vs baseline: 1.2724x; 1.0038x over previous
import jax
import jax.numpy as jnp
from jax.experimental import pallas as pl
from jax.experimental.pallas import tpu as pltpu

_BM = 512


def _sage_block(adj_ref, x_ref, wrel_ref, wroot_ref, b_ref, out_ref):
    i = pl.program_id(0)
    a = adj_ref[...].astype(jnp.bfloat16)
    deg = jnp.clip(jnp.sum(a, axis=1, keepdims=True, dtype=jnp.float32),
                   1.0, None)
    agg = jnp.dot(a, x_ref[...].astype(jnp.bfloat16),
                  preferred_element_type=jnp.float32)
    agg = agg / deg
    x_blk = x_ref[pl.ds(i * _BM, _BM), :]
    out = (jnp.dot(agg, wrel_ref[...], preferred_element_type=jnp.float32)
           + jnp.dot(x_blk, wroot_ref[...], preferred_element_type=jnp.float32)
           + b_ref[...])
    nrm = jnp.sqrt(jnp.sum(out * out, axis=1, keepdims=True))
    out = out / jnp.clip(nrm, 1e-12, None)
    out_ref[...] = jnp.where(out >= 0, out, 0.01 * out)


def kernel(x, adj, W_rel, W_root, b):
    B, N, C_in = x.shape
    C_out = W_rel.shape[1]
    x2 = x.reshape(N, C_in)
    adj2 = adj.reshape(N, N)
    b2 = b.reshape(1, C_out)
    out = pl.pallas_call(
        _sage_block,
        grid=(N // _BM,),
        in_specs=[
            pl.BlockSpec((_BM, N), lambda i: (i, 0)),
            pl.BlockSpec((N, C_in), lambda i: (0, 0)),
            pl.BlockSpec((C_in, C_out), lambda i: (0, 0)),
            pl.BlockSpec((C_in, C_out), lambda i: (0, 0)),
            pl.BlockSpec((1, C_out), lambda i: (0, 0)),
        ],
        out_specs=pl.BlockSpec((_BM, C_out), lambda i: (i, 0)),
        out_shape=jax.ShapeDtypeStruct((N, C_out), jnp.float32),
        compiler_params=pltpu.CompilerParams(
            dimension_semantics=("parallel",)),
    )(adj2, x2, W_rel, W_root, b2)
    return out.reshape(B, N, C_out)
